# Initial kernel scaffold; baseline (speedup 1.0000x reference)
import jax
import jax.numpy as jnp
from jax.experimental import pallas as pl


def kernel(h, e, edge_index, params):
    raise NotImplementedError("write your pallas kernel here")


# speedup vs baseline: 14.1293x; 14.1293x over previous
import functools

import jax
import jax.numpy as jnp
from jax import lax
from jax.experimental import pallas as pl
from jax.experimental.pallas import tpu as pltpu
from jax.experimental.pallas import tpu_sc as plsc

_N = 100000
_E = 1600000
_CFGS = [(16, 8, 32, 16, 2), (64, 32, 32, 16, 2), (64, 32, 64, 32, 1)]

_NC, _NS = 2, 16
_C = 512
_NCHUNK = _E // _C
_GPERW = -(-_NCHUNK // (_NC * _NS))
_SPERT = -(-_NCHUNK // _NS)
_RPT = _N // _NS
_ZC = 625

_F32 = jnp.float32


def _leaky(x):
    return jnp.where(x >= 0, x, 0.01 * x)


def _node_tables(h, pw, pb, wnis, wnjb, do_proj):
    B = 2000
    K = h.shape[1]

    def body(h_ref, pw_ref, pb_ref, wnis_ref, wnjb_ref, ftab_ref, fnj_ref):
        x = h_ref[...]
        if do_proj:
            x = _leaky(jnp.dot(x, pw_ref[...], preferred_element_type=_F32)
                       + pb_ref[...])
        ftab_ref[...] = jnp.dot(x, wnis_ref[...], preferred_element_type=_F32)
        fnj_ref[...] = (jnp.dot(x, wnjb_ref[...][:-1], preferred_element_type=_F32)
                        + wnjb_ref[...][-1:])

    kin = wnis.shape[0]
    return pl.pallas_call(
        body,
        grid=(_N // B,),
        in_specs=[
            pl.BlockSpec((B, K), lambda i: (i, 0)),
            pl.BlockSpec(pw.shape, lambda i: (0, 0)),
            pl.BlockSpec(pb.shape, lambda i: (0, 0)),
            pl.BlockSpec((kin, 96), lambda i: (0, 0)),
            pl.BlockSpec((kin + 1, 32), lambda i: (0, 0)),
        ],
        out_specs=[
            pl.BlockSpec((B, 96), lambda i: (i, 0)),
            pl.BlockSpec((B, 32), lambda i: (i, 0)),
        ],
        out_shape=[
            jax.ShapeDtypeStruct((_N, 96), _F32),
            jax.ShapeDtypeStruct((_N, 32), _F32),
        ],
    )(h, pw, pb, wnis, wnjb)


def _edge_math(e_cur, fsrcg, fnjg, wf, bf, abd, smat, pmat):
    B = 4000
    Ke = e_cur.shape[1]

    def body(e_ref, fs_ref, fn_ref, wf_ref, bf_ref, a_ref, s_ref, p_ref,
             fo_ref, v_ref):
        ffij = jnp.dot(e_ref[...], wf_ref[...], preferred_element_type=_F32)
        fo = _leaky(fs_ref[...][:, :32] + fn_ref[...] + ffij + bf_ref[...])
        fo_ref[...] = fo
        ex = jnp.exp(jnp.dot(fo, a_ref[...], preferred_element_type=_F32))
        scale = jnp.dot(ex, s_ref[...], preferred_element_type=_F32)
        vmain = fs_ref[...][:, 32:] * scale
        for b in range(4):
            v_ref[b] = vmain[:, 16 * b:16 * (b + 1)]
        v_ref[4] = jnp.dot(ex, p_ref[...], preferred_element_type=_F32)

    return pl.pallas_call(
        body,
        grid=(_E // B,),
        in_specs=[
            pl.BlockSpec((B, Ke), lambda i: (i, 0)),
            pl.BlockSpec((B, 96), lambda i: (i, 0)),
            pl.BlockSpec((B, 32), lambda i: (i, 0)),
            pl.BlockSpec((Ke, 32), lambda i: (0, 0)),
            pl.BlockSpec((1, 32), lambda i: (0, 0)),
            pl.BlockSpec((32, 8), lambda i: (0, 0)),
            pl.BlockSpec((8, 64), lambda i: (0, 0)),
            pl.BlockSpec((8, 16), lambda i: (0, 0)),
        ],
        out_specs=[
            pl.BlockSpec((B, 32), lambda i: (i, 0)),
            pl.BlockSpec((5, B, 16), lambda i: (0, i, 0)),
        ],
        out_shape=[
            jax.ShapeDtypeStruct((_E, 32), _F32),
            jax.ShapeDtypeStruct((5, _E, 16), _F32),
        ],
    )(e_cur, fsrcg, fnjg, wf, bf, abd, smat, pmat)


def _normalize(acc, heads):
    B = 2000

    def body(a_ref, o_ref):
        a = a_ref[...]
        num = jnp.concatenate([a[0], a[1], a[2], a[3]], axis=1)
        if heads == 2:
            d0 = jnp.broadcast_to(a[4][:, 0:1], (B, 32))
            d1 = jnp.broadcast_to(a[4][:, 1:2], (B, 32))
            den = jnp.concatenate([d0, d1], axis=1)
        else:
            den = jnp.broadcast_to(a[4][:, 0:1], (B, 64))
        o_ref[...] = _leaky(jnp.where(den > 0, num / den, 0.0))

    return pl.pallas_call(
        body,
        grid=(_N // B,),
        in_specs=[pl.BlockSpec((5, B, 16), lambda i: (0, i, 0))],
        out_specs=pl.BlockSpec((B, 64), lambda i: (i, 0)),
        out_shape=jax.ShapeDtypeStruct((_N, 64), _F32),
    )(acc)


def _readout(h, w1, b1, w2, b2, wp1, bp1, wp2, bp2):
    B = 2000
    nblk = _N // B

    def body(h_ref, w1_ref, b1_ref, w2_ref, b2_ref, wp1_ref, bp1_ref,
             wp2_ref, bp2_ref, p1_ref, p2_ref, acc):
        i = pl.program_id(0)

        @pl.when(i == 0)
        def _():
            acc[...] = jnp.zeros_like(acc)

        blk = jnp.sum(h_ref[...], axis=0, keepdims=True) / float(_N)
        acc[...] += jnp.broadcast_to(blk, acc.shape)

        @pl.when(i == nblk - 1)
        def _():
            hg = acc[...][0:1]
            x = _leaky(jnp.dot(hg, w1_ref[...], preferred_element_type=_F32)
                       + b1_ref[...])
            x = _leaky(jnp.dot(x, w2_ref[...], preferred_element_type=_F32)
                       + b2_ref[...])
            z1 = jnp.dot(x, wp1_ref[...], preferred_element_type=_F32) + bp1_ref[...]
            z2 = jnp.dot(x, wp2_ref[...], preferred_element_type=_F32) + bp2_ref[...]
            p1_ref[...] = 1.0 / (1.0 + jnp.exp(-z1))
            p2_ref[...] = 1.0 / (1.0 + jnp.exp(-z2))

    full = lambda a: pl.BlockSpec(a.shape, lambda i: tuple(0 for _ in a.shape))
    return pl.pallas_call(
        body,
        grid=(nblk,),
        in_specs=[pl.BlockSpec((B, 64), lambda i: (i, 0)),
                  full(w1), full(b1), full(w2), full(b2),
                  full(wp1), full(bp1), full(wp2), full(bp2)],
        out_specs=[pl.BlockSpec((1, 2), lambda i: (0, 0)),
                   pl.BlockSpec((1, 2), lambda i: (0, 0))],
        out_shape=[jax.ShapeDtypeStruct((1, 2), _F32),
                   jax.ShapeDtypeStruct((1, 2), _F32)],
        scratch_shapes=[pltpu.VMEM((8, 64), _F32)],
    )(h, w1, b1, w2, b2, wp1, bp1, wp2, bp2)


def _sc_mesh():
    return plsc.VectorSubcoreMesh(core_axis_name="c", subcore_axis_name="s",
                                  num_cores=_NC, num_subcores=_NS)


def _sc_gather(src, dst, ftab, fnj):
    @functools.partial(
        pl.kernel,
        out_type=[jax.ShapeDtypeStruct((_E, 96), _F32),
                  jax.ShapeDtypeStruct((_E, 32), _F32)],
        mesh=_sc_mesh(),
        scratch_types=[
            pltpu.VMEM((4, 128), jnp.int32),
            pltpu.VMEM((4, 128), jnp.int32),
            pltpu.VMEM((_C, 96), _F32),
            pltpu.VMEM((_C, 32), _F32),
            pltpu.SemaphoreType.DMA,
            pltpu.SemaphoreType.DMA,
        ],
        compiler_params=pltpu.CompilerParams(use_tc_tiling_on_sc=False),
    )
    def k(src_h, dst_h, ftab_h, fnj_h, og_h, on_h, idxs, idxd, bufs, bufd,
          sema, semb):
        wid = lax.axis_index("s") * _NC + lax.axis_index("c")

        def body(ci, carry):
            chunk = ci * (_NC * _NS) + wid

            @pl.when(chunk < _NCHUNK)
            def _():
                base = chunk * _C
                for j in range(4):
                    pltpu.sync_copy(src_h.at[pl.ds(base + j * 128, 128)],
                                    idxs.at[j])
                    pltpu.sync_copy(dst_h.at[pl.ds(base + j * 128, 128)],
                                    idxd.at[j])
                cps = []
                for j in range(4):
                    cps.append(pltpu.async_copy(
                        ftab_h.at[idxs.at[j]],
                        bufs.at[pl.ds(j * 128, 128)], sema))
                    cps.append(pltpu.async_copy(
                        fnj_h.at[idxd.at[j]],
                        bufd.at[pl.ds(j * 128, 128)], semb))
                for cp in cps:
                    cp.wait()
                pltpu.sync_copy(bufs, og_h.at[pl.ds(base, _C)])
                pltpu.sync_copy(bufd, on_h.at[pl.ds(base, _C)])

            return carry

        lax.fori_loop(0, _GPERW, body, 0)

    return k(src, dst, ftab, fnj)


def _sc_scatter(dst, v):
    @functools.partial(
        pl.kernel,
        out_type=jax.ShapeDtypeStruct((5, _N, 16), _F32),
        mesh=_sc_mesh(),
        scratch_types=[
            pltpu.VMEM((4, 128), jnp.int32),
            pltpu.VMEM((_C, 16), _F32),
            pltpu.VMEM((_ZC, 16), _F32),
            pltpu.VMEM((_ZC, 16), _F32),
            pltpu.VMEM_SHARED((_N, 16), _F32),
        ],
        compiler_params=pltpu.CompilerParams(use_tc_tiling_on_sc=False),
    )
    def k(dst_h, v_h, acc_h, idx, vals, zbuf, obuf, table):
        c = lax.axis_index("c")
        s = lax.axis_index("s")
        row0 = s * _RPT

        def zb(i, carry):
            zbuf[i] = jnp.zeros((16,), _F32)
            return carry

        lax.fori_loop(0, _ZC, zb, 0)

        def one_round(b):
            for kk in range(_RPT // _ZC):
                pltpu.sync_copy(zbuf, table.at[pl.ds(row0 + kk * _ZC, _ZC)])
            plsc.subcore_barrier()

            def body(ci, carry):
                chunk = ci * _NS + s

                @pl.when(chunk < _NCHUNK)
                def _():
                    base = chunk * _C
                    for j in range(4):
                        pltpu.sync_copy(dst_h.at[pl.ds(base + j * 128, 128)],
                                        idx.at[j])
                    pltpu.sync_copy(v_h.at[b, pl.ds(base, _C)], vals)
                    for j in range(4):
                        pltpu.sync_copy(vals.at[pl.ds(j * 128, 128)],
                                        table.at[idx.at[j]], add=True)

                return carry

            lax.fori_loop(0, _SPERT, body, 0)
            plsc.subcore_barrier()
            for kk in range(_RPT // _ZC):
                r0 = row0 + kk * _ZC
                pltpu.sync_copy(table.at[pl.ds(r0, _ZC)], obuf)
                pltpu.sync_copy(obuf, acc_h.at[b, pl.ds(r0, _ZC)])

        for r in range(3):
            @pl.when(c == 0)
            def _(r=r):
                one_round([0, 2, 4][r])

            if r < 2:
                @pl.when(c == 1)
                def _(r=r):
                    one_round([1, 3][r])

    return k(dst, v)


def kernel(h, e, edge_index, params):
    src = edge_index[0]
    dst = edge_index[1]
    hp = jnp.pad(h, ((0, 0), (0, 2)))
    pw = jnp.pad(params["proj_h_W"], ((0, 2), (0, 0)))
    pb = params["proj_h_b"].reshape(1, 16)
    ep = jnp.pad(e, ((0, 0), (0, 7)))

    h_cur = None
    e_cur = ep
    f_out = None
    for li, (p, (in_n, in_e, out_n, out_e, H)) in enumerate(zip(params["layers"], _CFGS)):
        wnis = jnp.concatenate([p["W_ni"], p["W_src"]], axis=1)
        wnjb = jnp.concatenate([p["W_nj"], p["b_e"].reshape(1, 32)], axis=0)
        if li == 0:
            ftab, fnj = _node_tables(hp, pw, pb, wnis, wnjb, True)
            wf = jnp.pad(params["proj_e_W"] @ p["W_fij"], ((0, 7), (0, 0)))
            bf = (params["proj_e_b"].reshape(1, 8) @ p["W_fij"]).reshape(1, 32)
        else:
            ftab, fnj = _node_tables(h_cur, pw, pb, wnis, wnjb, False)
            wf = p["W_fij"]
            bf = jnp.zeros((1, 32), _F32)

        abd = jnp.zeros((32, 8), _F32)
        for hh in range(H):
            abd = abd.at[hh * out_e:(hh + 1) * out_e, hh].set(p["attn"][hh])
        smat = jnp.zeros((8, 64), _F32)
        for hh in range(H):
            smat = smat.at[hh, hh * out_n:(hh + 1) * out_n].set(1.0)
        pmat = jnp.zeros((8, 16), _F32)
        for hh in range(H):
            pmat = pmat.at[hh, hh].set(1.0)

        fsrcg, fnjg = _sc_gather(src, dst, ftab, fnj)
        f_out, v = _edge_math(e_cur, fsrcg, fnjg, wf, bf, abd, smat, pmat)
        acc = _sc_scatter(dst, v)
        h_cur = _normalize(acc, H)
        e_cur = f_out

    pr = params
    return tuple(_readout(
        h_cur,
        pr["pred_W1"], pr["pred_b1"].reshape(1, 16),
        pr["pred_W2"], pr["pred_b2"].reshape(1, 8),
        pr["pred_Wp1"], pr["pred_bp1"].reshape(1, 2),
        pr["pred_Wp2"], pr["pred_bp2"].reshape(1, 2)))


# speedup vs baseline: 25.3739x; 1.7958x over previous
import functools

import jax
import jax.numpy as jnp
from jax import lax
from jax.experimental import pallas as pl
from jax.experimental.pallas import tpu as pltpu
from jax.experimental.pallas import tpu_sc as plsc

_N = 100000
_E = 1600000
_CFGS = [(16, 8, 32, 16, 2), (64, 32, 32, 16, 2), (64, 32, 64, 32, 1)]

_NC, _NS = 2, 16
_NW = _NC * _NS

_GC = 256
_GNCH = _E // _GC
_GPERW = -(-_GNCH // _NW)

_SC = 512
_SNCH = _E // _SC
_SPERT = -(-_SNCH // _NS)
_RPT = _N // _NS
_ZC = 625

_F32 = jnp.float32


def _leaky(x):
    return jnp.where(x >= 0, x, 0.01 * x)


def _node_table(h, pw, pb, wall, brow, do_proj):
    B = 2000
    K = h.shape[1]

    def body(h_ref, pw_ref, pb_ref, w_ref, b_ref, o_ref):
        x = h_ref[...]
        if do_proj:
            x = _leaky(jnp.dot(x, pw_ref[...], preferred_element_type=_F32)
                       + pb_ref[...])
        o_ref[...] = (jnp.dot(x, w_ref[...], preferred_element_type=_F32)
                      + b_ref[...])

    kin = wall.shape[0]
    return pl.pallas_call(
        body,
        grid=(_N // B,),
        in_specs=[
            pl.BlockSpec((B, K), lambda i: (i, 0)),
            pl.BlockSpec(pw.shape, lambda i: (0, 0)),
            pl.BlockSpec(pb.shape, lambda i: (0, 0)),
            pl.BlockSpec((kin, 128), lambda i: (0, 0)),
            pl.BlockSpec((1, 128), lambda i: (0, 0)),
        ],
        out_specs=pl.BlockSpec((B, 128), lambda i: (i, 0)),
        out_shape=jax.ShapeDtypeStruct((_N, 128), _F32),
    )(h, pw, pb, wall, brow)


def _edge_math(e_cur, og, wf, bf, abd, smat, pmat):
    B = 8000
    Ke = e_cur.shape[1]

    def body(e_ref, og_ref, wf_ref, bf_ref, a_ref, s_ref, p_ref,
             fo_ref, v_ref):
        g = og_ref[...]
        ffij = jnp.dot(e_ref[...], wf_ref[...], preferred_element_type=_F32)
        fo = _leaky(g[:, :32] + ffij + bf_ref[...])
        fo_ref[...] = fo
        ex = jnp.exp(jnp.dot(fo, a_ref[...], preferred_element_type=_F32))
        scale = jnp.dot(ex, s_ref[...], preferred_element_type=_F32)
        vmain = g[:, 32:96] * scale
        vden = jnp.dot(ex, p_ref[...], preferred_element_type=_F32)
        v_ref[...] = jnp.concatenate(
            [vmain, vden, jnp.zeros((B, 48), _F32)], axis=1)

    return pl.pallas_call(
        body,
        grid=(_E // B,),
        in_specs=[
            pl.BlockSpec((B, Ke), lambda i: (i, 0)),
            pl.BlockSpec((B, 128), lambda i: (i, 0)),
            pl.BlockSpec((Ke, 32), lambda i: (0, 0)),
            pl.BlockSpec((1, 32), lambda i: (0, 0)),
            pl.BlockSpec((32, 8), lambda i: (0, 0)),
            pl.BlockSpec((8, 64), lambda i: (0, 0)),
            pl.BlockSpec((8, 16), lambda i: (0, 0)),
        ],
        out_specs=[
            pl.BlockSpec((B, 32), lambda i: (i, 0)),
            pl.BlockSpec((B, 128), lambda i: (i, 0)),
        ],
        out_shape=[
            jax.ShapeDtypeStruct((_E, 32), _F32),
            jax.ShapeDtypeStruct((_E, 128), _F32),
        ],
    )(e_cur, og, wf, bf, abd, smat, pmat)


def _normalize(acc, heads):
    B = 2000

    def body(a_ref, o_ref):
        a = a_ref[...]
        num = a[:, :64]
        if heads == 2:
            d0 = jnp.broadcast_to(a[:, 64:65], (B, 32))
            d1 = jnp.broadcast_to(a[:, 65:66], (B, 32))
            den = jnp.concatenate([d0, d1], axis=1)
        else:
            den = jnp.broadcast_to(a[:, 64:65], (B, 64))
        o_ref[...] = _leaky(jnp.where(den > 0, num / den, 0.0))

    return pl.pallas_call(
        body,
        grid=(_N // B,),
        in_specs=[pl.BlockSpec((B, 128), lambda i: (i, 0))],
        out_specs=pl.BlockSpec((B, 64), lambda i: (i, 0)),
        out_shape=jax.ShapeDtypeStruct((_N, 64), _F32),
    )(acc)


def _readout(h, w1, b1, w2, b2, wp1, bp1, wp2, bp2):
    B = 2000
    nblk = _N // B

    def body(h_ref, w1_ref, b1_ref, w2_ref, b2_ref, wp1_ref, bp1_ref,
             wp2_ref, bp2_ref, p1_ref, p2_ref, acc):
        i = pl.program_id(0)

        @pl.when(i == 0)
        def _():
            acc[...] = jnp.zeros_like(acc)

        blk = jnp.sum(h_ref[...], axis=0, keepdims=True) / float(_N)
        acc[...] += jnp.broadcast_to(blk, acc.shape)

        @pl.when(i == nblk - 1)
        def _():
            hg = acc[...][0:1]
            x = _leaky(jnp.dot(hg, w1_ref[...], preferred_element_type=_F32)
                       + b1_ref[...])
            x = _leaky(jnp.dot(x, w2_ref[...], preferred_element_type=_F32)
                       + b2_ref[...])
            z1 = jnp.dot(x, wp1_ref[...], preferred_element_type=_F32) + bp1_ref[...]
            z2 = jnp.dot(x, wp2_ref[...], preferred_element_type=_F32) + bp2_ref[...]
            p1_ref[...] = 1.0 / (1.0 + jnp.exp(-z1))
            p2_ref[...] = 1.0 / (1.0 + jnp.exp(-z2))

    full = lambda a: pl.BlockSpec(a.shape, lambda i: tuple(0 for _ in a.shape))
    return pl.pallas_call(
        body,
        grid=(nblk,),
        in_specs=[pl.BlockSpec((B, 64), lambda i: (i, 0)),
                  full(w1), full(b1), full(w2), full(b2),
                  full(wp1), full(bp1), full(wp2), full(bp2)],
        out_specs=[pl.BlockSpec((1, 2), lambda i: (0, 0)),
                   pl.BlockSpec((1, 2), lambda i: (0, 0))],
        out_shape=[jax.ShapeDtypeStruct((1, 2), _F32),
                   jax.ShapeDtypeStruct((1, 2), _F32)],
        scratch_shapes=[pltpu.VMEM((8, 64), _F32)],
    )(h, w1, b1, w2, b2, wp1, bp1, wp2, bp2)


def _sc_mesh():
    return plsc.VectorSubcoreMesh(core_axis_name="c", subcore_axis_name="s",
                                  num_cores=_NC, num_subcores=_NS)


def _sc_gather(src, dst, ntab):
    @functools.partial(
        pl.kernel,
        out_type=jax.ShapeDtypeStruct((_E, 128), _F32),
        mesh=_sc_mesh(),
        scratch_types=[
            pltpu.VMEM((8, 128), jnp.int32),
            pltpu.VMEM((_GC, 128), _F32),
            pltpu.VMEM((_GC, 128), _F32),
            pltpu.SemaphoreType.DMA,
            pltpu.SemaphoreType.DMA,
        ],
    )
    def k(src_h, dst_h, ntab_h, og_h, idx, bufa, bufb, sema, semb):
        wid = lax.axis_index("s") * _NC + lax.axis_index("c")

        def body(ci, carry):
            chunk = ci * _NW + wid

            @pl.when(chunk < _GNCH)
            def _():
                base = chunk * _GC
                for j in range(2):
                    pltpu.sync_copy(src_h.at[pl.ds(base + j * 128, 128)],
                                    idx.at[j])
                    pltpu.sync_copy(dst_h.at[pl.ds(base + j * 128, 128)],
                                    idx.at[2 + j])
                cps = []
                for j in range(2):
                    cps.append(pltpu.async_copy(
                        ntab_h.at[idx.at[j]],
                        bufa.at[pl.ds(j * 128, 128)], sema))
                    cps.append(pltpu.async_copy(
                        ntab_h.at[idx.at[2 + j]],
                        bufb.at[pl.ds(j * 128, 128)], semb))
                for cp in cps:
                    cp.wait()

                def addrow(r, carry2):
                    bufa[r, pl.ds(0, 16)] += bufb[r, pl.ds(96, 16)]
                    bufa[r, pl.ds(16, 16)] += bufb[r, pl.ds(112, 16)]
                    return carry2

                lax.fori_loop(0, _GC, addrow, 0)
                pltpu.sync_copy(bufa, og_h.at[pl.ds(base, _GC)])

            return carry

        lax.fori_loop(0, _GPERW, body, 0)

    return k(src, dst, ntab)


def _sc_scatter(dst, v128):
    @functools.partial(
        pl.kernel,
        out_type=jax.ShapeDtypeStruct((_N, 128), _F32),
        mesh=_sc_mesh(),
        scratch_types=[
            pltpu.VMEM((4, 128), jnp.int32),
            pltpu.VMEM((_SC, 16), _F32),
            pltpu.VMEM((_ZC, 16), _F32),
            pltpu.VMEM((_ZC, 16), _F32),
            pltpu.VMEM_SHARED((_N, 16), _F32),
        ],
        compiler_params=pltpu.CompilerParams(use_tc_tiling_on_sc=False),
    )
    def k(dst_h, v_h, acc_h, idx, vals, zbuf, obuf, table):
        c = lax.axis_index("c")
        s = lax.axis_index("s")
        row0 = s * _RPT

        def zb(i, carry):
            zbuf[i] = jnp.zeros((16,), _F32)
            return carry

        lax.fori_loop(0, _ZC, zb, 0)

        def one_round(b):
            for kk in range(_RPT // _ZC):
                pltpu.sync_copy(zbuf, table.at[pl.ds(row0 + kk * _ZC, _ZC)])
            plsc.subcore_barrier()

            def body(ci, carry):
                chunk = ci * _NS + s

                @pl.when(chunk < _SNCH)
                def _():
                    base = chunk * _SC
                    for j in range(4):
                        pltpu.sync_copy(dst_h.at[pl.ds(base + j * 128, 128)],
                                        idx.at[j])
                    pltpu.sync_copy(
                        v_h.at[pl.ds(base, _SC), pl.ds(16 * b, 16)], vals)
                    for j in range(4):
                        pltpu.sync_copy(vals.at[pl.ds(j * 128, 128)],
                                        table.at[idx.at[j]], add=True)

                return carry

            lax.fori_loop(0, _SPERT, body, 0)
            plsc.subcore_barrier()
            for kk in range(_RPT // _ZC):
                r0 = row0 + kk * _ZC
                pltpu.sync_copy(table.at[pl.ds(r0, _ZC)], obuf)
                pltpu.sync_copy(obuf,
                                acc_h.at[pl.ds(r0, _ZC), pl.ds(16 * b, 16)])

        for r in range(3):
            @pl.when(c == 0)
            def _(r=r):
                one_round([0, 2, 4][r])

            if r < 2:
                @pl.when(c == 1)
                def _(r=r):
                    one_round([1, 3][r])

    return k(dst, v128)


def kernel(h, e, edge_index, params):
    src = edge_index[0]
    dst = edge_index[1]
    hp = jnp.pad(h, ((0, 0), (0, 2)))
    pw = jnp.pad(params["proj_h_W"], ((0, 2), (0, 0)))
    pb = params["proj_h_b"].reshape(1, 16)
    ep = jnp.pad(e, ((0, 0), (0, 7)))

    h_cur = None
    e_cur = ep
    for li, (p, (in_n, in_e, out_n, out_e, H)) in enumerate(zip(params["layers"], _CFGS)):
        wall = jnp.concatenate([p["W_ni"], p["W_src"], p["W_nj"]], axis=1)
        brow = jnp.concatenate([jnp.zeros((1, 96), _F32),
                                p["b_e"].reshape(1, 32)], axis=1)
        if li == 0:
            ntab = _node_table(hp, pw, pb, wall, brow, True)
            wf = jnp.pad(params["proj_e_W"] @ p["W_fij"], ((0, 7), (0, 0)))
            bf = (params["proj_e_b"].reshape(1, 8) @ p["W_fij"]).reshape(1, 32)
        else:
            ntab = _node_table(h_cur, pw, pb, wall, brow, False)
            wf = p["W_fij"]
            bf = jnp.zeros((1, 32), _F32)

        abd = jnp.zeros((32, 8), _F32)
        for hh in range(H):
            abd = abd.at[hh * out_e:(hh + 1) * out_e, hh].set(p["attn"][hh])
        smat = jnp.zeros((8, 64), _F32)
        for hh in range(H):
            smat = smat.at[hh, hh * out_n:(hh + 1) * out_n].set(1.0)
        pmat = jnp.zeros((8, 16), _F32)
        for hh in range(H):
            pmat = pmat.at[hh, hh].set(1.0)

        og = _sc_gather(src, dst, ntab)
        f_out, v128 = _edge_math(e_cur, og, wf, bf, abd, smat, pmat)
        acc = _sc_scatter(dst, v128)
        h_cur = _normalize(acc, H)
        e_cur = f_out

    pr = params
    return tuple(_readout(
        h_cur,
        pr["pred_W1"], pr["pred_b1"].reshape(1, 16),
        pr["pred_W2"], pr["pred_b2"].reshape(1, 8),
        pr["pred_Wp1"], pr["pred_bp1"].reshape(1, 2),
        pr["pred_Wp2"], pr["pred_bp2"].reshape(1, 2)))


# speedup vs baseline: 36.5830x; 1.4418x over previous
import functools

import jax
import jax.numpy as jnp
from jax import lax
from jax.experimental import pallas as pl
from jax.experimental.pallas import tpu as pltpu
from jax.experimental.pallas import tpu_sc as plsc

_N = 100000
_E = 1600000
_CFGS = [(16, 8, 32, 16, 2), (64, 32, 32, 16, 2), (64, 32, 64, 32, 1)]

_NC, _NS = 2, 16
_NW = _NC * _NS

_GC = 256
_GNCH = _E // _GC
_GPERW = -(-_GNCH // _NW)

_SC = 640
_SNCH = _E // _SC
_SPERT = -(-_SNCH // _NS)
_RPT = _N // _NS
_ZC = 250

_F32 = jnp.float32


def _leaky(x):
    return jnp.where(x >= 0, x, 0.01 * x)


def _node_table(h, pw, pb, wall, brow, do_proj):
    B = 2000
    K = h.shape[1]

    def body(h_ref, pw_ref, pb_ref, w_ref, b_ref, o_ref):
        x = h_ref[...]
        if do_proj:
            x = _leaky(jnp.dot(x, pw_ref[...], preferred_element_type=_F32)
                       + pb_ref[...])
        o_ref[...] = (jnp.dot(x, w_ref[...], preferred_element_type=_F32)
                      + b_ref[...])

    kin = wall.shape[0]
    return pl.pallas_call(
        body,
        grid=(_N // B,),
        in_specs=[
            pl.BlockSpec((B, K), lambda i: (i, 0)),
            pl.BlockSpec(pw.shape, lambda i: (0, 0)),
            pl.BlockSpec(pb.shape, lambda i: (0, 0)),
            pl.BlockSpec((kin, 128), lambda i: (0, 0)),
            pl.BlockSpec((1, 128), lambda i: (0, 0)),
        ],
        out_specs=pl.BlockSpec((B, 128), lambda i: (i, 0)),
        out_shape=jax.ShapeDtypeStruct((_N, 128), _F32),
    )(h, pw, pb, wall, brow)


def _edge_math(e_cur, og, wf, bf, abd, smat, pmat):
    B = 8000
    Ke = e_cur.shape[1]

    def body(e_ref, og_ref, wf_ref, bf_ref, a_ref, s_ref, p_ref,
             fo_ref, v_ref):
        g = og_ref[...]
        ffij = jnp.dot(e_ref[...], wf_ref[...], preferred_element_type=_F32)
        fo = _leaky(g[:, :32] + ffij + bf_ref[...])
        fo_ref[...] = fo
        ex = jnp.exp(jnp.dot(fo, a_ref[...], preferred_element_type=_F32))
        scale = jnp.dot(ex, s_ref[...], preferred_element_type=_F32)
        vmain = g[:, 32:96] * scale
        vden = jnp.dot(ex, p_ref[...], preferred_element_type=_F32)
        v_ref[...] = jnp.concatenate(
            [vmain, vden, jnp.zeros((B, 48), _F32)], axis=1)

    return pl.pallas_call(
        body,
        grid=(_E // B,),
        in_specs=[
            pl.BlockSpec((B, Ke), lambda i: (i, 0)),
            pl.BlockSpec((B, 128), lambda i: (i, 0)),
            pl.BlockSpec((Ke, 32), lambda i: (0, 0)),
            pl.BlockSpec((1, 32), lambda i: (0, 0)),
            pl.BlockSpec((32, 8), lambda i: (0, 0)),
            pl.BlockSpec((8, 64), lambda i: (0, 0)),
            pl.BlockSpec((8, 16), lambda i: (0, 0)),
        ],
        out_specs=[
            pl.BlockSpec((B, 32), lambda i: (i, 0)),
            pl.BlockSpec((B, 128), lambda i: (i, 0)),
        ],
        out_shape=[
            jax.ShapeDtypeStruct((_E, 32), _F32),
            jax.ShapeDtypeStruct((_E, 128), _F32),
        ],
    )(e_cur, og, wf, bf, abd, smat, pmat)


def _normalize(acc, heads):
    B = 2000

    def body(a_ref, o_ref):
        a = a_ref[...]
        num = a[:, :64]
        dh = a[:, 64:80] + a[:, 80:96]
        if heads == 2:
            d0 = jnp.broadcast_to(dh[:, 0:1], (B, 32))
            d1 = jnp.broadcast_to(dh[:, 1:2], (B, 32))
            den = jnp.concatenate([d0, d1], axis=1)
        else:
            den = jnp.broadcast_to(dh[:, 0:1], (B, 64))
        o_ref[...] = _leaky(jnp.where(den > 0, num / den, 0.0))

    return pl.pallas_call(
        body,
        grid=(_N // B,),
        in_specs=[pl.BlockSpec((B, 128), lambda i: (i, 0))],
        out_specs=pl.BlockSpec((B, 64), lambda i: (i, 0)),
        out_shape=jax.ShapeDtypeStruct((_N, 64), _F32),
    )(acc)


def _readout(h, w1, b1, w2, b2, wp1, bp1, wp2, bp2):
    B = 2000
    nblk = _N // B

    def body(h_ref, w1_ref, b1_ref, w2_ref, b2_ref, wp1_ref, bp1_ref,
             wp2_ref, bp2_ref, p1_ref, p2_ref, acc):
        i = pl.program_id(0)

        @pl.when(i == 0)
        def _():
            acc[...] = jnp.zeros_like(acc)

        blk = jnp.sum(h_ref[...], axis=0, keepdims=True) / float(_N)
        acc[...] += jnp.broadcast_to(blk, acc.shape)

        @pl.when(i == nblk - 1)
        def _():
            hg = acc[...][0:1]
            x = _leaky(jnp.dot(hg, w1_ref[...], preferred_element_type=_F32)
                       + b1_ref[...])
            x = _leaky(jnp.dot(x, w2_ref[...], preferred_element_type=_F32)
                       + b2_ref[...])
            z1 = jnp.dot(x, wp1_ref[...], preferred_element_type=_F32) + bp1_ref[...]
            z2 = jnp.dot(x, wp2_ref[...], preferred_element_type=_F32) + bp2_ref[...]
            p1_ref[...] = 1.0 / (1.0 + jnp.exp(-z1))
            p2_ref[...] = 1.0 / (1.0 + jnp.exp(-z2))

    full = lambda a: pl.BlockSpec(a.shape, lambda i: tuple(0 for _ in a.shape))
    return pl.pallas_call(
        body,
        grid=(nblk,),
        in_specs=[pl.BlockSpec((B, 64), lambda i: (i, 0)),
                  full(w1), full(b1), full(w2), full(b2),
                  full(wp1), full(bp1), full(wp2), full(bp2)],
        out_specs=[pl.BlockSpec((1, 2), lambda i: (0, 0)),
                   pl.BlockSpec((1, 2), lambda i: (0, 0))],
        out_shape=[jax.ShapeDtypeStruct((1, 2), _F32),
                   jax.ShapeDtypeStruct((1, 2), _F32)],
        scratch_shapes=[pltpu.VMEM((8, 64), _F32)],
    )(h, w1, b1, w2, b2, wp1, bp1, wp2, bp2)


def _split_edges(edge_index):
    Br = _E // 128

    def body(ei_ref, s_ref, d_ref):
        s_ref[...] = ei_ref[0].reshape(Br, 128)
        d_ref[...] = ei_ref[1].reshape(Br, 128)

    return pl.pallas_call(
        body,
        out_shape=[jax.ShapeDtypeStruct((Br, 128), jnp.int32),
                   jax.ShapeDtypeStruct((Br, 128), jnp.int32)],
    )(edge_index)


def _sc_mesh():
    return plsc.VectorSubcoreMesh(core_axis_name="c", subcore_axis_name="s",
                                  num_cores=_NC, num_subcores=_NS)


def _sc_gather(src, dst, ntab):
    @functools.partial(
        pl.kernel,
        out_type=jax.ShapeDtypeStruct((_E, 128), _F32),
        mesh=_sc_mesh(),
        scratch_types=[
            pltpu.VMEM((8, 128), jnp.int32),
            pltpu.VMEM((_GC, 128), _F32),
            pltpu.VMEM((_GC, 128), _F32),
            pltpu.SemaphoreType.DMA,
            pltpu.SemaphoreType.DMA,
            pltpu.SemaphoreType.DMA,
        ],
    )
    def k(src_h, dst_h, ntab_h, og_h, idx, bufa, bufb, sema, semb, semi):
        wid = lax.axis_index("s") * _NC + lax.axis_index("c")

        def body(ci, carry):
            chunk = ci * _NW + wid

            @pl.when(chunk < _GNCH)
            def _():
                base = chunk * _GC
                row = chunk * 2
                cps = []
                for j in range(2):
                    cps.append(pltpu.async_copy(
                        src_h.at[row + j], idx.at[j], semi))
                    cps.append(pltpu.async_copy(
                        dst_h.at[row + j], idx.at[2 + j], semi))
                for cp in cps:
                    cp.wait()
                cps = []
                for j in range(2):
                    cps.append(pltpu.async_copy(
                        ntab_h.at[idx.at[j]],
                        bufa.at[pl.ds(j * 128, 128)], sema))
                    cps.append(pltpu.async_copy(
                        ntab_h.at[idx.at[2 + j]],
                        bufb.at[pl.ds(j * 128, 128)], semb))
                for cp in cps:
                    cp.wait()

                def addrow(r, carry2):
                    bufa[r, pl.ds(0, 16)] += bufb[r, pl.ds(96, 16)]
                    bufa[r, pl.ds(16, 16)] += bufb[r, pl.ds(112, 16)]
                    return carry2

                lax.fori_loop(0, _GC, addrow, 0)
                pltpu.sync_copy(bufa, og_h.at[pl.ds(base, _GC)])

            return carry

        lax.fori_loop(0, _GPERW, body, 0)

    return k(src, dst, ntab)


def _sc_scatter(dst, v128):
    nsub = _SC // 128

    @functools.partial(
        pl.kernel,
        out_type=jax.ShapeDtypeStruct((_N, 128), _F32),
        mesh=_sc_mesh(),
        scratch_types=[
            pltpu.VMEM((8, 128), jnp.int32),
            pltpu.VMEM((_SC, 16), _F32),
            pltpu.VMEM((_ZC, 16), _F32),
            pltpu.VMEM((_ZC, 16), _F32),
            pltpu.VMEM_SHARED((_N, 16), _F32),
            pltpu.SemaphoreType.DMA,
            pltpu.SemaphoreType.DMA,
        ],
        compiler_params=pltpu.CompilerParams(use_tc_tiling_on_sc=False),
    )
    def k(dst_h, v_h, acc_h, idx, vals, zbuf, obuf, table, semi, sems):
        c = lax.axis_index("c")
        s = lax.axis_index("s")
        row0 = s * _RPT

        def zb(i, carry):
            zbuf[i] = jnp.zeros((16,), _F32)
            return carry

        lax.fori_loop(0, _ZC, zb, 0)

        def one_round(b, lo, hi, ocol):
            for kk in range(_RPT // _ZC):
                pltpu.sync_copy(zbuf, table.at[pl.ds(row0 + kk * _ZC, _ZC)])
            plsc.subcore_barrier()

            def body(ci, carry):
                chunk = ci * _NS + s

                @pl.when(jnp.logical_and(chunk >= lo, chunk < hi))
                def _():
                    base = chunk * _SC
                    row = chunk * nsub
                    cps = []
                    for j in range(nsub):
                        cps.append(pltpu.async_copy(
                            dst_h.at[row + j], idx.at[j], semi))
                    pltpu.sync_copy(
                        v_h.at[pl.ds(base, _SC), pl.ds(16 * b, 16)], vals)
                    for cp in cps:
                        cp.wait()
                    cps = []
                    for j in range(nsub):
                        cps.append(pltpu.async_copy(
                            vals.at[pl.ds(j * 128, 128)],
                            table.at[idx.at[j]], sems, add=True))
                    for cp in cps:
                        cp.wait()

                return carry

            lax.fori_loop(0, _SPERT, body, 0)
            plsc.subcore_barrier()
            for kk in range(_RPT // _ZC):
                r0 = row0 + kk * _ZC
                pltpu.sync_copy(table.at[pl.ds(r0, _ZC)], obuf)
                pltpu.sync_copy(obuf, acc_h.at[pl.ds(r0, _ZC),
                                               pl.ds(ocol, 16)])

        half = _SNCH // 2
        for r in range(3):
            @pl.when(c == 0)
            def _(r=r):
                if r < 2:
                    one_round([0, 2][r], 0, _SNCH, 16 * [0, 2][r])
                else:
                    one_round(4, 0, half, 64)

            @pl.when(c == 1)
            def _(r=r):
                if r < 2:
                    one_round([1, 3][r], 0, _SNCH, 16 * [1, 3][r])
                else:
                    one_round(4, half, _SNCH, 80)

    return k(dst, v128)


def kernel(h, e, edge_index, params):
    src, dst = _split_edges(edge_index)
    hp = jnp.pad(h, ((0, 0), (0, 2)))
    pw = jnp.pad(params["proj_h_W"], ((0, 2), (0, 0)))
    pb = params["proj_h_b"].reshape(1, 16)
    ep = jnp.pad(e, ((0, 0), (0, 7)))

    h_cur = None
    e_cur = ep
    for li, (p, (in_n, in_e, out_n, out_e, H)) in enumerate(zip(params["layers"], _CFGS)):
        wall = jnp.concatenate([p["W_ni"], p["W_src"], p["W_nj"]], axis=1)
        brow = jnp.concatenate([jnp.zeros((1, 96), _F32),
                                p["b_e"].reshape(1, 32)], axis=1)
        if li == 0:
            ntab = _node_table(hp, pw, pb, wall, brow, True)
            wf = jnp.pad(params["proj_e_W"] @ p["W_fij"], ((0, 7), (0, 0)))
            bf = (params["proj_e_b"].reshape(1, 8) @ p["W_fij"]).reshape(1, 32)
        else:
            ntab = _node_table(h_cur, pw, pb, wall, brow, False)
            wf = p["W_fij"]
            bf = jnp.zeros((1, 32), _F32)

        abd = jnp.zeros((32, 8), _F32)
        for hh in range(H):
            abd = abd.at[hh * out_e:(hh + 1) * out_e, hh].set(p["attn"][hh])
        smat = jnp.zeros((8, 64), _F32)
        for hh in range(H):
            smat = smat.at[hh, hh * out_n:(hh + 1) * out_n].set(1.0)
        pmat = jnp.zeros((8, 16), _F32)
        for hh in range(H):
            pmat = pmat.at[hh, hh].set(1.0)

        og = _sc_gather(src, dst, ntab)
        f_out, v128 = _edge_math(e_cur, og, wf, bf, abd, smat, pmat)
        acc = _sc_scatter(dst, v128)
        h_cur = _normalize(acc, H)
        e_cur = f_out

    pr = params
    return tuple(_readout(
        h_cur,
        pr["pred_W1"], pr["pred_b1"].reshape(1, 16),
        pr["pred_W2"], pr["pred_b2"].reshape(1, 8),
        pr["pred_Wp1"], pr["pred_bp1"].reshape(1, 2),
        pr["pred_Wp2"], pr["pred_bp2"].reshape(1, 2)))


# speedup vs baseline: 47.6578x; 1.3027x over previous
import functools

import jax
import jax.numpy as jnp
from jax import lax
from jax.experimental import pallas as pl
from jax.experimental.pallas import tpu as pltpu
from jax.experimental.pallas import tpu_sc as plsc

_N = 100000
_E = 1600000
_CFGS = [(16, 8, 32, 16, 2), (64, 32, 32, 16, 2), (64, 32, 64, 32, 1)]

_NC, _NS = 2, 16
_NW = _NC * _NS

_EH = _E // 2

_GC = 256
_GNCH = _EH // _GC
_GPERW = -(-_GNCH // _NW)

_SC = 640
_SNCH = _EH // _SC
_SPERT = -(-_SNCH // _NS)
_RPT = _N // _NS
_ZC = 250

_F32 = jnp.float32


def _leaky(x):
    return jnp.where(x >= 0, x, 0.01 * x)


def _node_table(h, pw, pb, wall, brow, do_proj):
    B = 2000
    K = h.shape[1]

    def body(h_ref, pw_ref, pb_ref, w_ref, b_ref, o_ref):
        x = h_ref[...]
        if do_proj:
            x = _leaky(jnp.dot(x, pw_ref[...], preferred_element_type=_F32)
                       + pb_ref[...])
        o_ref[...] = (jnp.dot(x, w_ref[...], preferred_element_type=_F32)
                      + b_ref[...])

    kin = wall.shape[0]
    return pl.pallas_call(
        body,
        grid=(_N // B,),
        in_specs=[
            pl.BlockSpec((B, K), lambda i: (i, 0)),
            pl.BlockSpec(pw.shape, lambda i: (0, 0)),
            pl.BlockSpec(pb.shape, lambda i: (0, 0)),
            pl.BlockSpec((kin, 128), lambda i: (0, 0)),
            pl.BlockSpec((1, 128), lambda i: (0, 0)),
        ],
        out_specs=pl.BlockSpec((B, 128), lambda i: (i, 0)),
        out_shape=jax.ShapeDtypeStruct((_N, 128), _F32),
    )(h, pw, pb, wall, brow)


def _edge_math(e_cur, og, wf, bf, abd, smat, pmat):
    B = 8000

    def body(*refs):
        if e_cur is None:
            og_ref, bf_ref, a_ref, s_ref, p_ref, fo_ref, v_ref = refs
        else:
            (e_ref, og_ref, wf_ref, bf_ref, a_ref, s_ref, p_ref,
             fo_ref, v_ref) = refs
        g = og_ref[...]
        t = g[:, :32] + bf_ref[...]
        if e_cur is not None:
            t = t + jnp.dot(e_ref[...], wf_ref[...],
                            preferred_element_type=_F32)
        fo = _leaky(t)
        fo_ref[...] = fo
        ex = jnp.exp(jnp.dot(fo, a_ref[...], preferred_element_type=_F32))
        scale = jnp.dot(ex, s_ref[...], preferred_element_type=_F32)
        vmain = g[:, 32:96] * scale
        vden = jnp.dot(ex, p_ref[...], preferred_element_type=_F32)
        v_ref[...] = jnp.concatenate(
            [vmain, vden, jnp.zeros((B, 48), _F32)], axis=1)

    in_specs = [
        pl.BlockSpec((B, 128), lambda i: (i, 0)),
        pl.BlockSpec((1, 32), lambda i: (0, 0)),
        pl.BlockSpec((32, 8), lambda i: (0, 0)),
        pl.BlockSpec((8, 64), lambda i: (0, 0)),
        pl.BlockSpec((8, 16), lambda i: (0, 0)),
    ]
    args = [og, bf, abd, smat, pmat]
    if e_cur is not None:
        Ke = e_cur.shape[1]
        in_specs = [pl.BlockSpec((B, Ke), lambda i: (i, 0)),
                    in_specs[0],
                    pl.BlockSpec((Ke, 32), lambda i: (0, 0))] + in_specs[1:]
        args = [e_cur, og, wf] + args[1:]

    return pl.pallas_call(
        body,
        grid=(_EH // B,),
        in_specs=in_specs,
        out_specs=[
            pl.BlockSpec((B, 32), lambda i: (i, 0)),
            pl.BlockSpec((B, 128), lambda i: (i, 0)),
        ],
        out_shape=[
            jax.ShapeDtypeStruct((_EH, 32), _F32),
            jax.ShapeDtypeStruct((_EH, 128), _F32),
        ],
    )(*args)


def _normalize(acca, accb, heads):
    B = 2000

    def body(aa_ref, ab_ref, o_ref):
        a = aa_ref[...] + ab_ref[...]
        num = a[:, :64]
        dh = a[:, 64:80] + a[:, 80:96]
        if heads == 2:
            d0 = jnp.broadcast_to(dh[:, 0:1], (B, 32))
            d1 = jnp.broadcast_to(dh[:, 1:2], (B, 32))
            den = jnp.concatenate([d0, d1], axis=1)
        else:
            den = jnp.broadcast_to(dh[:, 0:1], (B, 64))
        o_ref[...] = _leaky(jnp.where(den > 0, num / den, 0.0))

    return pl.pallas_call(
        body,
        grid=(_N // B,),
        in_specs=[pl.BlockSpec((B, 128), lambda i: (i, 0)),
                  pl.BlockSpec((B, 128), lambda i: (i, 0))],
        out_specs=pl.BlockSpec((B, 64), lambda i: (i, 0)),
        out_shape=jax.ShapeDtypeStruct((_N, 64), _F32),
    )(acca, accb)


def _readout(h, w1, b1, w2, b2, wp1, bp1, wp2, bp2):
    B = 2000
    nblk = _N // B

    def body(h_ref, w1_ref, b1_ref, w2_ref, b2_ref, wp1_ref, bp1_ref,
             wp2_ref, bp2_ref, p1_ref, p2_ref, acc):
        i = pl.program_id(0)

        @pl.when(i == 0)
        def _():
            acc[...] = jnp.zeros_like(acc)

        blk = jnp.sum(h_ref[...], axis=0, keepdims=True) / float(_N)
        acc[...] += jnp.broadcast_to(blk, acc.shape)

        @pl.when(i == nblk - 1)
        def _():
            hg = acc[...][0:1]
            x = _leaky(jnp.dot(hg, w1_ref[...], preferred_element_type=_F32)
                       + b1_ref[...])
            x = _leaky(jnp.dot(x, w2_ref[...], preferred_element_type=_F32)
                       + b2_ref[...])
            z1 = jnp.dot(x, wp1_ref[...], preferred_element_type=_F32) + bp1_ref[...]
            z2 = jnp.dot(x, wp2_ref[...], preferred_element_type=_F32) + bp2_ref[...]
            p1_ref[...] = 1.0 / (1.0 + jnp.exp(-z1))
            p2_ref[...] = 1.0 / (1.0 + jnp.exp(-z2))

    full = lambda a: pl.BlockSpec(a.shape, lambda i: tuple(0 for _ in a.shape))
    return pl.pallas_call(
        body,
        grid=(nblk,),
        in_specs=[pl.BlockSpec((B, 64), lambda i: (i, 0)),
                  full(w1), full(b1), full(w2), full(b2),
                  full(wp1), full(bp1), full(wp2), full(bp2)],
        out_specs=[pl.BlockSpec((1, 2), lambda i: (0, 0)),
                   pl.BlockSpec((1, 2), lambda i: (0, 0))],
        out_shape=[jax.ShapeDtypeStruct((1, 2), _F32),
                   jax.ShapeDtypeStruct((1, 2), _F32)],
        scratch_shapes=[pltpu.VMEM((8, 64), _F32)],
    )(h, w1, b1, w2, b2, wp1, bp1, wp2, bp2)


def _split_edges(edge_index):
    Br = _E // 128

    def body(ei_ref, s_ref, d_ref):
        s_ref[...] = ei_ref[0].reshape(Br, 128)
        d_ref[...] = ei_ref[1].reshape(Br, 128)

    return pl.pallas_call(
        body,
        out_shape=[jax.ShapeDtypeStruct((Br, 128), jnp.int32),
                   jax.ShapeDtypeStruct((Br, 128), jnp.int32)],
    )(edge_index)


def _sc_mesh():
    return plsc.VectorSubcoreMesh(core_axis_name="c", subcore_axis_name="s",
                                  num_cores=_NC, num_subcores=_NS)


def _sc_gather(src2, dst2, ntab, half, e1=None, ew=None):
    lo = half * _GNCH
    with_e = e1 is not None

    scratch = [
        pltpu.VMEM((8, 128), jnp.int32),
        pltpu.VMEM((_GC, 128), _F32),
        pltpu.VMEM((_GC, 128), _F32),
        pltpu.SemaphoreType.DMA,
        pltpu.SemaphoreType.DMA,
        pltpu.SemaphoreType.DMA,
    ]
    if with_e:
        scratch = scratch + [pltpu.VMEM((_GC + 16,), _F32),
                             pltpu.VMEM((128,), _F32)]

    @functools.partial(
        pl.kernel,
        out_type=jax.ShapeDtypeStruct((_EH, 128), _F32),
        mesh=_sc_mesh(),
        scratch_types=scratch,
    )
    def k(*refs):
        if with_e:
            (src_h, dst_h, ntab_h, e_h, ew_h, og_h,
             idx, bufa, bufb, sema, semb, semi, ebuf, wbuf) = refs
            pltpu.sync_copy(ew_h, wbuf)
        else:
            (src_h, dst_h, ntab_h, og_h,
             idx, bufa, bufb, sema, semb, semi) = refs
        wid = lax.axis_index("s") * _NC + lax.axis_index("c")

        def body(ci, carry):
            chunk = lo + ci * _NW + wid

            @pl.when(chunk < lo + _GNCH)
            def _():
                base = chunk * _GC
                row = chunk * 2
                cps = []
                for j in range(2):
                    cps.append(pltpu.async_copy(
                        src_h.at[row + j], idx.at[j], semi))
                    cps.append(pltpu.async_copy(
                        dst_h.at[row + j], idx.at[2 + j], semi))
                if with_e:
                    cps.append(pltpu.async_copy(
                        e_h.at[pl.ds(base, _GC)],
                        ebuf.at[pl.ds(0, _GC)], semi))
                for cp in cps:
                    cp.wait()
                cps = []
                for j in range(2):
                    cps.append(pltpu.async_copy(
                        ntab_h.at[idx.at[j]],
                        bufa.at[pl.ds(j * 128, 128)], sema))
                    cps.append(pltpu.async_copy(
                        ntab_h.at[idx.at[2 + j]],
                        bufb.at[pl.ds(j * 128, 128)], semb))
                for cp in cps:
                    cp.wait()

                def addrow(r, carry2):
                    if with_e:
                        ev = ebuf[pl.ds(r, 16)][0]
                        bufa[r, pl.ds(0, 16)] += (bufb[r, pl.ds(96, 16)]
                                                  + ev * wbuf[pl.ds(0, 16)])
                        bufa[r, pl.ds(16, 16)] += (bufb[r, pl.ds(112, 16)]
                                                   + ev * wbuf[pl.ds(16, 16)])
                    else:
                        bufa[r, pl.ds(0, 16)] += bufb[r, pl.ds(96, 16)]
                        bufa[r, pl.ds(16, 16)] += bufb[r, pl.ds(112, 16)]
                    return carry2

                lax.fori_loop(0, _GC, addrow, 0)
                pltpu.sync_copy(bufa, og_h.at[pl.ds(base - lo * _GC, _GC)])

            return carry

        lax.fori_loop(0, _GPERW, body, 0)

    if with_e:
        return k(src2, dst2, ntab, e1, ew)
    return k(src2, dst2, ntab)


def _sc_scatter(dst2, v128, half):
    nsub = _SC // 128
    clo = half * _SNCH

    @functools.partial(
        pl.kernel,
        out_type=jax.ShapeDtypeStruct((_N, 128), _F32),
        mesh=_sc_mesh(),
        scratch_types=[
            pltpu.VMEM((8, 128), jnp.int32),
            pltpu.VMEM((_SC, 16), _F32),
            pltpu.VMEM((_ZC, 16), _F32),
            pltpu.VMEM((_ZC, 16), _F32),
            pltpu.VMEM_SHARED((_N, 16), _F32),
            pltpu.SemaphoreType.DMA,
            pltpu.SemaphoreType.DMA,
        ],
        compiler_params=pltpu.CompilerParams(use_tc_tiling_on_sc=False),
    )
    def k(dst_h, v_h, acc_h, idx, vals, zbuf, obuf, table, semi, sems):
        c = lax.axis_index("c")
        s = lax.axis_index("s")
        row0 = s * _RPT

        def zb(i, carry):
            zbuf[i] = jnp.zeros((16,), _F32)
            return carry

        lax.fori_loop(0, _ZC, zb, 0)

        def one_round(b, lo, hi, ocol):
            for kk in range(_RPT // _ZC):
                pltpu.sync_copy(zbuf, table.at[pl.ds(row0 + kk * _ZC, _ZC)])
            plsc.subcore_barrier()

            def body(ci, carry):
                chunk = clo + ci * _NS + s

                @pl.when(jnp.logical_and(chunk >= lo, chunk < hi))
                def _():
                    base = (chunk - clo) * _SC
                    row = chunk * nsub
                    cps = []
                    for j in range(nsub):
                        cps.append(pltpu.async_copy(
                            dst_h.at[row + j], idx.at[j], semi))
                    pltpu.sync_copy(
                        v_h.at[pl.ds(base, _SC), pl.ds(16 * b, 16)], vals)
                    for cp in cps:
                        cp.wait()
                    cps = []
                    for j in range(nsub):
                        cps.append(pltpu.async_copy(
                            vals.at[pl.ds(j * 128, 128)],
                            table.at[idx.at[j]], sems, add=True))
                    for cp in cps:
                        cp.wait()

                return carry

            lax.fori_loop(0, _SPERT, body, 0)
            plsc.subcore_barrier()
            for kk in range(_RPT // _ZC):
                r0 = row0 + kk * _ZC
                pltpu.sync_copy(table.at[pl.ds(r0, _ZC)], obuf)
                pltpu.sync_copy(obuf, acc_h.at[pl.ds(r0, _ZC),
                                               pl.ds(ocol, 16)])

        mid = clo + _SNCH // 2
        for r in range(3):
            @pl.when(c == 0)
            def _(r=r):
                if r < 2:
                    one_round([0, 2][r], clo, clo + _SNCH, 16 * [0, 2][r])
                else:
                    one_round(4, clo, mid, 64)

            @pl.when(c == 1)
            def _(r=r):
                if r < 2:
                    one_round([1, 3][r], clo, clo + _SNCH, 16 * [1, 3][r])
                else:
                    one_round(4, mid, clo + _SNCH, 80)

    return k(dst2, v128)


def kernel(h, e, edge_index, params):
    src2, dst2 = _split_edges(edge_index)
    hp = jnp.pad(h, ((0, 0), (0, 2)))
    pw = jnp.pad(params["proj_h_W"], ((0, 2), (0, 0)))
    pb = params["proj_h_b"].reshape(1, 16)
    e1 = e.reshape(_E)

    h_cur = None
    e_half = [None, None]
    for li, (p, (in_n, in_e, out_n, out_e, H)) in enumerate(zip(params["layers"], _CFGS)):
        wall = jnp.concatenate([p["W_ni"], p["W_src"], p["W_nj"]], axis=1)
        brow = jnp.concatenate([jnp.zeros((1, 96), _F32),
                                p["b_e"].reshape(1, 32)], axis=1)
        if li == 0:
            ntab = _node_table(hp, pw, pb, wall, brow, True)
            ew = jnp.pad((params["proj_e_W"] @ p["W_fij"]).reshape(32), (0, 96))
            bf = (params["proj_e_b"].reshape(1, 8) @ p["W_fij"]).reshape(1, 32)
            wf = None
        else:
            ntab = _node_table(h_cur, pw, pb, wall, brow, False)
            ew = None
            wf = p["W_fij"]
            bf = jnp.zeros((1, 32), _F32)

        abd = jnp.zeros((32, 8), _F32)
        for hh in range(H):
            abd = abd.at[hh * out_e:(hh + 1) * out_e, hh].set(p["attn"][hh])
        smat = jnp.zeros((8, 64), _F32)
        for hh in range(H):
            smat = smat.at[hh, hh * out_n:(hh + 1) * out_n].set(1.0)
        pmat = jnp.zeros((8, 16), _F32)
        for hh in range(H):
            pmat = pmat.at[hh, hh].set(1.0)

        accs = [None, None]
        new_e = [None, None]
        for hf in range(2):
            if li == 0:
                og = _sc_gather(src2, dst2, ntab, hf, e1, ew)
            else:
                og = _sc_gather(src2, dst2, ntab, hf)
            f_out, v128 = _edge_math(e_half[hf], og, wf, bf, abd, smat, pmat)
            accs[hf] = _sc_scatter(dst2, v128, hf)
            new_e[hf] = f_out
        h_cur = _normalize(accs[0], accs[1], H)
        e_half = new_e

    pr = params
    return tuple(_readout(
        h_cur,
        pr["pred_W1"], pr["pred_b1"].reshape(1, 16),
        pr["pred_W2"], pr["pred_b2"].reshape(1, 8),
        pr["pred_Wp1"], pr["pred_bp1"].reshape(1, 2),
        pr["pred_Wp2"], pr["pred_bp2"].reshape(1, 2)))


# speedup vs baseline: 49.0502x; 1.0292x over previous
import functools

import jax
import jax.numpy as jnp
from jax import lax
from jax.experimental import pallas as pl
from jax.experimental.pallas import tpu as pltpu
from jax.experimental.pallas import tpu_sc as plsc

_N = 100000
_E = 1600000
_CFGS = [(16, 8, 32, 16, 2), (64, 32, 32, 16, 2), (64, 32, 64, 32, 1)]

_NC, _NS = 2, 16
_NW = _NC * _NS

_EH = _E // 2

_GC = 256
_GNCH = _EH // _GC
_GPERW = -(-_GNCH // _NW)

_SC = 640
_SNCH = _EH // _SC
_SPERT = -(-_SNCH // _NS)
_RPT = _N // _NS
_ZC = 250

_F32 = jnp.float32


def _leaky(x):
    return jnp.where(x >= 0, x, 0.01 * x)


def _node_table(h, pw, pb, wall, brow, do_proj):
    B = 2000
    K = h.shape[1]

    def body(h_ref, pw_ref, pb_ref, w_ref, b_ref, o_ref):
        x = h_ref[...]
        if do_proj:
            x = _leaky(jnp.dot(x, pw_ref[...], preferred_element_type=_F32)
                       + pb_ref[...])
        o_ref[...] = (jnp.dot(x, w_ref[...], preferred_element_type=_F32)
                      + b_ref[...])

    kin = wall.shape[0]
    return pl.pallas_call(
        body,
        grid=(_N // B,),
        in_specs=[
            pl.BlockSpec((B, K), lambda i: (i, 0)),
            pl.BlockSpec(pw.shape, lambda i: (0, 0)),
            pl.BlockSpec(pb.shape, lambda i: (0, 0)),
            pl.BlockSpec((kin, 128), lambda i: (0, 0)),
            pl.BlockSpec((1, 128), lambda i: (0, 0)),
        ],
        out_specs=pl.BlockSpec((B, 128), lambda i: (i, 0)),
        out_shape=jax.ShapeDtypeStruct((_N, 128), _F32),
    )(h, pw, pb, wall, brow)


def _edge_math(e_cur, og, wf, bf, abd, smat, pmat):
    B = 8000

    def body(*refs):
        if e_cur is None:
            og_ref, bf_ref, a_ref, s_ref, p_ref, fo_ref, v_ref = refs
        else:
            (e_ref, og_ref, wf_ref, bf_ref, a_ref, s_ref, p_ref,
             fo_ref, v_ref) = refs
        g = og_ref[...]
        t = g[:, :32] + bf_ref[...]
        if e_cur is not None:
            t = t + jnp.dot(e_ref[...], wf_ref[...],
                            preferred_element_type=_F32)
        fo = _leaky(t)
        fo_ref[...] = fo
        ex = jnp.exp(jnp.dot(fo, a_ref[...], preferred_element_type=_F32))
        scale = jnp.dot(ex, s_ref[...], preferred_element_type=_F32)
        vmain = g[:, 32:96] * scale
        vden = jnp.dot(ex, p_ref[...], preferred_element_type=_F32)
        v_ref[...] = jnp.concatenate(
            [vmain, vden, jnp.zeros((B, 48), _F32)], axis=1)

    in_specs = [
        pl.BlockSpec((B, 128), lambda i: (i, 0)),
        pl.BlockSpec((1, 32), lambda i: (0, 0)),
        pl.BlockSpec((32, 8), lambda i: (0, 0)),
        pl.BlockSpec((8, 64), lambda i: (0, 0)),
        pl.BlockSpec((8, 16), lambda i: (0, 0)),
    ]
    args = [og, bf, abd, smat, pmat]
    if e_cur is not None:
        Ke = e_cur.shape[1]
        in_specs = [pl.BlockSpec((B, Ke), lambda i: (i, 0)),
                    in_specs[0],
                    pl.BlockSpec((Ke, 32), lambda i: (0, 0))] + in_specs[1:]
        args = [e_cur, og, wf] + args[1:]

    return pl.pallas_call(
        body,
        grid=(_EH // B,),
        in_specs=in_specs,
        out_specs=[
            pl.BlockSpec((B, 32), lambda i: (i, 0)),
            pl.BlockSpec((B, 128), lambda i: (i, 0)),
        ],
        out_shape=[
            jax.ShapeDtypeStruct((_EH, 32), _F32),
            jax.ShapeDtypeStruct((_EH, 128), _F32),
        ],
    )(*args)


def _normalize(acca, accb, heads):
    B = 2000

    def body(aa_ref, ab_ref, o_ref):
        a = aa_ref[...] + ab_ref[...]
        num = a[:, :64]
        dh = a[:, 64:80] + a[:, 80:96]
        if heads == 2:
            d0 = jnp.broadcast_to(dh[:, 0:1], (B, 32))
            d1 = jnp.broadcast_to(dh[:, 1:2], (B, 32))
            den = jnp.concatenate([d0, d1], axis=1)
        else:
            den = jnp.broadcast_to(dh[:, 0:1], (B, 64))
        o_ref[...] = _leaky(jnp.where(den > 0, num / den, 0.0))

    return pl.pallas_call(
        body,
        grid=(_N // B,),
        in_specs=[pl.BlockSpec((B, 128), lambda i: (i, 0)),
                  pl.BlockSpec((B, 128), lambda i: (i, 0))],
        out_specs=pl.BlockSpec((B, 64), lambda i: (i, 0)),
        out_shape=jax.ShapeDtypeStruct((_N, 64), _F32),
    )(acca, accb)


def _readout(h, w1, b1, w2, b2, wp1, bp1, wp2, bp2):
    B = 2000
    nblk = _N // B

    def body(h_ref, w1_ref, b1_ref, w2_ref, b2_ref, wp1_ref, bp1_ref,
             wp2_ref, bp2_ref, p1_ref, p2_ref, acc):
        i = pl.program_id(0)

        @pl.when(i == 0)
        def _():
            acc[...] = jnp.zeros_like(acc)

        blk = jnp.sum(h_ref[...], axis=0, keepdims=True) / float(_N)
        acc[...] += jnp.broadcast_to(blk, acc.shape)

        @pl.when(i == nblk - 1)
        def _():
            hg = acc[...][0:1]
            x = _leaky(jnp.dot(hg, w1_ref[...], preferred_element_type=_F32)
                       + b1_ref[...])
            x = _leaky(jnp.dot(x, w2_ref[...], preferred_element_type=_F32)
                       + b2_ref[...])
            z1 = jnp.dot(x, wp1_ref[...], preferred_element_type=_F32) + bp1_ref[...]
            z2 = jnp.dot(x, wp2_ref[...], preferred_element_type=_F32) + bp2_ref[...]
            p1_ref[...] = 1.0 / (1.0 + jnp.exp(-z1))
            p2_ref[...] = 1.0 / (1.0 + jnp.exp(-z2))

    full = lambda a: pl.BlockSpec(a.shape, lambda i: tuple(0 for _ in a.shape))
    return pl.pallas_call(
        body,
        grid=(nblk,),
        in_specs=[pl.BlockSpec((B, 64), lambda i: (i, 0)),
                  full(w1), full(b1), full(w2), full(b2),
                  full(wp1), full(bp1), full(wp2), full(bp2)],
        out_specs=[pl.BlockSpec((1, 2), lambda i: (0, 0)),
                   pl.BlockSpec((1, 2), lambda i: (0, 0))],
        out_shape=[jax.ShapeDtypeStruct((1, 2), _F32),
                   jax.ShapeDtypeStruct((1, 2), _F32)],
        scratch_shapes=[pltpu.VMEM((8, 64), _F32)],
    )(h, w1, b1, w2, b2, wp1, bp1, wp2, bp2)


def _split_edges(edge_index):
    Br = _E // 128

    def body(ei_ref, s_ref, d_ref):
        s_ref[...] = ei_ref[0].reshape(Br, 128)
        d_ref[...] = ei_ref[1].reshape(Br, 128)

    return pl.pallas_call(
        body,
        out_shape=[jax.ShapeDtypeStruct((Br, 128), jnp.int32),
                   jax.ShapeDtypeStruct((Br, 128), jnp.int32)],
    )(edge_index)


def _sc_mesh():
    return plsc.VectorSubcoreMesh(core_axis_name="c", subcore_axis_name="s",
                                  num_cores=_NC, num_subcores=_NS)


def _sc_gather(src2, dst2, ntab, half, e1=None, ew=None):
    lo = half * _GNCH
    with_e = e1 is not None

    scratch = [
        pltpu.VMEM((8, 128), jnp.int32),
        pltpu.VMEM((_GC, 128), _F32),
        pltpu.VMEM((_GC, 128), _F32),
        pltpu.SemaphoreType.DMA,
        pltpu.SemaphoreType.DMA,
        pltpu.SemaphoreType.DMA,
        pltpu.SemaphoreType.DMA,
    ]
    if with_e:
        scratch = scratch + [pltpu.VMEM((_GC + 16,), _F32),
                             pltpu.VMEM((128,), _F32)]

    @functools.partial(
        pl.kernel,
        out_type=jax.ShapeDtypeStruct((_EH, 128), _F32),
        mesh=_sc_mesh(),
        scratch_types=scratch,
    )
    def k(*refs):
        if with_e:
            (src_h, dst_h, ntab_h, e_h, ew_h, og_h,
             idx, bufa, bufb, sema, semb, semi, semo, ebuf, wbuf) = refs
            pltpu.sync_copy(ew_h, wbuf)
        else:
            (src_h, dst_h, ntab_h, og_h,
             idx, bufa, bufb, sema, semb, semi, semo) = refs
        wid = lax.axis_index("s") * _NC + lax.axis_index("c")

        def body(ci, carry):
            chunk = lo + ci * _NW + wid

            @pl.when(chunk < lo + _GNCH)
            def _():
                base = chunk * _GC
                row = chunk * 2
                cps = []
                for j in range(2):
                    cps.append(pltpu.async_copy(
                        src_h.at[row + j], idx.at[j], semi))
                    cps.append(pltpu.async_copy(
                        dst_h.at[row + j], idx.at[2 + j], semi))
                if with_e:
                    cps.append(pltpu.async_copy(
                        e_h.at[pl.ds(base, _GC)],
                        ebuf.at[pl.ds(0, _GC)], semi))
                for cp in cps:
                    cp.wait()

                @pl.when(ci > 0)
                def _():
                    pltpu.make_async_copy(
                        bufa, og_h.at[pl.ds(0, _GC)], semo).wait()

                cps = []
                for j in range(2):
                    cps.append(pltpu.async_copy(
                        ntab_h.at[idx.at[j]],
                        bufa.at[pl.ds(j * 128, 128)], sema))
                    cps.append(pltpu.async_copy(
                        ntab_h.at[idx.at[2 + j]],
                        bufb.at[pl.ds(j * 128, 128)], semb))
                for cp in cps:
                    cp.wait()

                def addrow(r, carry2):
                    if with_e:
                        ev = ebuf[pl.ds(r, 16)][0]
                        bufa[r, pl.ds(0, 16)] += (bufb[r, pl.ds(96, 16)]
                                                  + ev * wbuf[pl.ds(0, 16)])
                        bufa[r, pl.ds(16, 16)] += (bufb[r, pl.ds(112, 16)]
                                                   + ev * wbuf[pl.ds(16, 16)])
                    else:
                        bufa[r, pl.ds(0, 16)] += bufb[r, pl.ds(96, 16)]
                        bufa[r, pl.ds(16, 16)] += bufb[r, pl.ds(112, 16)]
                    return carry2

                lax.fori_loop(0, _GC, addrow, 0)
                pltpu.async_copy(bufa, og_h.at[pl.ds(base - lo * _GC, _GC)],
                                 semo)

            return carry

        lax.fori_loop(0, _GPERW, body, 0)
        pltpu.make_async_copy(bufa, og_h.at[pl.ds(0, _GC)], semo).wait()

    if with_e:
        return k(src2, dst2, ntab, e1, ew)
    return k(src2, dst2, ntab)


def _sc_scatter(dst2, v128, half):
    nsub = _SC // 128
    clo = half * _SNCH

    @functools.partial(
        pl.kernel,
        out_type=jax.ShapeDtypeStruct((_N, 128), _F32),
        mesh=_sc_mesh(),
        scratch_types=[
            pltpu.VMEM((8, 128), jnp.int32),
            pltpu.VMEM((_SC, 16), _F32),
            pltpu.VMEM((_ZC, 16), _F32),
            pltpu.VMEM((_ZC, 16), _F32),
            pltpu.VMEM_SHARED((_N, 16), _F32),
            pltpu.SemaphoreType.DMA,
            pltpu.SemaphoreType.DMA,
        ],
        compiler_params=pltpu.CompilerParams(use_tc_tiling_on_sc=False),
    )
    def k(dst_h, v_h, acc_h, idx, vals, zbuf, obuf, table, semi, sems):
        c = lax.axis_index("c")
        s = lax.axis_index("s")
        row0 = s * _RPT

        def zb(i, carry):
            zbuf[i] = jnp.zeros((16,), _F32)
            return carry

        lax.fori_loop(0, _ZC, zb, 0)

        def one_round(b, lo, hi, ocol):
            for kk in range(_RPT // _ZC):
                pltpu.sync_copy(zbuf, table.at[pl.ds(row0 + kk * _ZC, _ZC)])
            plsc.subcore_barrier()

            def body(ci, carry):
                chunk = clo + ci * _NS + s

                @pl.when(jnp.logical_and(chunk >= lo, chunk < hi))
                def _():
                    base = (chunk - clo) * _SC
                    row = chunk * nsub
                    cps = []
                    for j in range(nsub):
                        cps.append(pltpu.async_copy(
                            dst_h.at[row + j], idx.at[j], semi))
                    pltpu.sync_copy(
                        v_h.at[pl.ds(base, _SC), pl.ds(16 * b, 16)], vals)
                    for cp in cps:
                        cp.wait()
                    cps = []
                    for j in range(nsub):
                        cps.append(pltpu.async_copy(
                            vals.at[pl.ds(j * 128, 128)],
                            table.at[idx.at[j]], sems, add=True))
                    for cp in cps:
                        cp.wait()

                return carry

            lax.fori_loop(0, _SPERT, body, 0)
            plsc.subcore_barrier()
            for kk in range(_RPT // _ZC):
                r0 = row0 + kk * _ZC
                pltpu.sync_copy(table.at[pl.ds(r0, _ZC)], obuf)
                pltpu.sync_copy(obuf, acc_h.at[pl.ds(r0, _ZC),
                                               pl.ds(ocol, 16)])

        mid = clo + _SNCH // 2
        for r in range(3):
            @pl.when(c == 0)
            def _(r=r):
                if r < 2:
                    one_round([0, 2][r], clo, clo + _SNCH, 16 * [0, 2][r])
                else:
                    one_round(4, clo, mid, 64)

            @pl.when(c == 1)
            def _(r=r):
                if r < 2:
                    one_round([1, 3][r], clo, clo + _SNCH, 16 * [1, 3][r])
                else:
                    one_round(4, mid, clo + _SNCH, 80)

    return k(dst2, v128)


def kernel(h, e, edge_index, params):
    src2, dst2 = _split_edges(edge_index)
    hp = jnp.pad(h, ((0, 0), (0, 2)))
    pw = jnp.pad(params["proj_h_W"], ((0, 2), (0, 0)))
    pb = params["proj_h_b"].reshape(1, 16)
    e1 = e.reshape(_E)

    h_cur = None
    e_half = [None, None]
    for li, (p, (in_n, in_e, out_n, out_e, H)) in enumerate(zip(params["layers"], _CFGS)):
        wall = jnp.concatenate([p["W_ni"], p["W_src"], p["W_nj"]], axis=1)
        brow = jnp.concatenate([jnp.zeros((1, 96), _F32),
                                p["b_e"].reshape(1, 32)], axis=1)
        if li == 0:
            ntab = _node_table(hp, pw, pb, wall, brow, True)
            ew = jnp.pad((params["proj_e_W"] @ p["W_fij"]).reshape(32), (0, 96))
            bf = (params["proj_e_b"].reshape(1, 8) @ p["W_fij"]).reshape(1, 32)
            wf = None
        else:
            ntab = _node_table(h_cur, pw, pb, wall, brow, False)
            ew = None
            wf = p["W_fij"]
            bf = jnp.zeros((1, 32), _F32)

        abd = jnp.zeros((32, 8), _F32)
        for hh in range(H):
            abd = abd.at[hh * out_e:(hh + 1) * out_e, hh].set(p["attn"][hh])
        smat = jnp.zeros((8, 64), _F32)
        for hh in range(H):
            smat = smat.at[hh, hh * out_n:(hh + 1) * out_n].set(1.0)
        pmat = jnp.zeros((8, 16), _F32)
        for hh in range(H):
            pmat = pmat.at[hh, hh].set(1.0)

        accs = [None, None]
        new_e = [None, None]
        for hf in range(2):
            if li == 0:
                og = _sc_gather(src2, dst2, ntab, hf, e1, ew)
            else:
                og = _sc_gather(src2, dst2, ntab, hf)
            f_out, v128 = _edge_math(e_half[hf], og, wf, bf, abd, smat, pmat)
            accs[hf] = _sc_scatter(dst2, v128, hf)
            new_e[hf] = f_out
        h_cur = _normalize(accs[0], accs[1], H)
        e_half = new_e

    pr = params
    return tuple(_readout(
        h_cur,
        pr["pred_W1"], pr["pred_b1"].reshape(1, 16),
        pr["pred_W2"], pr["pred_b2"].reshape(1, 8),
        pr["pred_Wp1"], pr["pred_bp1"].reshape(1, 2),
        pr["pred_Wp2"], pr["pred_bp2"].reshape(1, 2)))


# speedup vs baseline: 50.8705x; 1.0371x over previous
import functools

import jax
import jax.numpy as jnp
from jax import lax
from jax.experimental import pallas as pl
from jax.experimental.pallas import tpu as pltpu
from jax.experimental.pallas import tpu_sc as plsc

_N = 100000
_E = 1600000
_CFGS = [(16, 8, 32, 16, 2), (64, 32, 32, 16, 2), (64, 32, 64, 32, 1)]

_NC, _NS = 2, 16
_NW = _NC * _NS

_EH = _E // 2

_GC = 256
_GNCH = _EH // _GC
_GPERW = -(-_GNCH // _NW)

_SC = 640
_SNCH = _EH // _SC
_SPERT = -(-_SNCH // _NS)
_RPT = _N // _NS
_ZC = 250

_F32 = jnp.float32


def _leaky(x):
    return jnp.where(x >= 0, x, 0.01 * x)


def _node_table(h, pw, pb, wall, brow, do_proj):
    B = 2000
    K = h.shape[1]

    def body(h_ref, pw_ref, pb_ref, w_ref, b_ref, o_ref, o2_ref):
        x = h_ref[...]
        if do_proj:
            x = _leaky(jnp.dot(x, pw_ref[...], preferred_element_type=_F32)
                       + pb_ref[...])
        t = (jnp.dot(x, w_ref[...], preferred_element_type=_F32)
             + b_ref[...])
        o_ref[...] = t
        o2_ref[...] = t[:, 96:128]

    kin = wall.shape[0]
    return pl.pallas_call(
        body,
        grid=(_N // B,),
        in_specs=[
            pl.BlockSpec((B, K), lambda i: (i, 0)),
            pl.BlockSpec(pw.shape, lambda i: (0, 0)),
            pl.BlockSpec(pb.shape, lambda i: (0, 0)),
            pl.BlockSpec((kin, 128), lambda i: (0, 0)),
            pl.BlockSpec((1, 128), lambda i: (0, 0)),
        ],
        out_specs=[pl.BlockSpec((B, 128), lambda i: (i, 0)),
                   pl.BlockSpec((B, 32), lambda i: (i, 0))],
        out_shape=[jax.ShapeDtypeStruct((_N, 128), _F32),
                   jax.ShapeDtypeStruct((_N, 32), _F32)],
    )(h, pw, pb, wall, brow)


def _edge_math(e_cur, og, wf, bf, abd, smat, pmat):
    B = 8000

    def body(*refs):
        if e_cur is None:
            og_ref, bf_ref, a_ref, s_ref, p_ref, fo_ref, v_ref = refs
        else:
            (e_ref, og_ref, wf_ref, bf_ref, a_ref, s_ref, p_ref,
             fo_ref, v_ref) = refs
        g = og_ref[...]
        t = g[:, :32] + bf_ref[...]
        if e_cur is not None:
            t = t + jnp.dot(e_ref[...], wf_ref[...],
                            preferred_element_type=_F32)
        fo = _leaky(t)
        fo_ref[...] = fo
        ex = jnp.exp(jnp.dot(fo, a_ref[...], preferred_element_type=_F32))
        scale = jnp.dot(ex, s_ref[...], preferred_element_type=_F32)
        vmain = g[:, 32:96] * scale
        vden = jnp.dot(ex, p_ref[...], preferred_element_type=_F32)
        v_ref[...] = jnp.concatenate(
            [vmain, vden, jnp.zeros((B, 48), _F32)], axis=1)

    in_specs = [
        pl.BlockSpec((B, 128), lambda i: (i, 0)),
        pl.BlockSpec((1, 32), lambda i: (0, 0)),
        pl.BlockSpec((32, 8), lambda i: (0, 0)),
        pl.BlockSpec((8, 64), lambda i: (0, 0)),
        pl.BlockSpec((8, 16), lambda i: (0, 0)),
    ]
    args = [og, bf, abd, smat, pmat]
    if e_cur is not None:
        Ke = e_cur.shape[1]
        in_specs = [pl.BlockSpec((B, Ke), lambda i: (i, 0)),
                    in_specs[0],
                    pl.BlockSpec((Ke, 32), lambda i: (0, 0))] + in_specs[1:]
        args = [e_cur, og, wf] + args[1:]

    return pl.pallas_call(
        body,
        grid=(_EH // B,),
        in_specs=in_specs,
        out_specs=[
            pl.BlockSpec((B, 32), lambda i: (i, 0)),
            pl.BlockSpec((B, 128), lambda i: (i, 0)),
        ],
        out_shape=[
            jax.ShapeDtypeStruct((_EH, 32), _F32),
            jax.ShapeDtypeStruct((_EH, 128), _F32),
        ],
    )(*args)


def _normalize(acca, accb, heads):
    B = 2000

    def body(aa_ref, ab_ref, o_ref):
        a = aa_ref[...] + ab_ref[...]
        num = a[:, :64]
        dh = a[:, 64:80] + a[:, 80:96]
        if heads == 2:
            d0 = jnp.broadcast_to(dh[:, 0:1], (B, 32))
            d1 = jnp.broadcast_to(dh[:, 1:2], (B, 32))
            den = jnp.concatenate([d0, d1], axis=1)
        else:
            den = jnp.broadcast_to(dh[:, 0:1], (B, 64))
        o_ref[...] = _leaky(jnp.where(den > 0, num / den, 0.0))

    return pl.pallas_call(
        body,
        grid=(_N // B,),
        in_specs=[pl.BlockSpec((B, 128), lambda i: (i, 0)),
                  pl.BlockSpec((B, 128), lambda i: (i, 0))],
        out_specs=pl.BlockSpec((B, 64), lambda i: (i, 0)),
        out_shape=jax.ShapeDtypeStruct((_N, 64), _F32),
    )(acca, accb)


def _readout(h, w1, b1, w2, b2, wp1, bp1, wp2, bp2):
    B = 2000
    nblk = _N // B

    def body(h_ref, w1_ref, b1_ref, w2_ref, b2_ref, wp1_ref, bp1_ref,
             wp2_ref, bp2_ref, p1_ref, p2_ref, acc):
        i = pl.program_id(0)

        @pl.when(i == 0)
        def _():
            acc[...] = jnp.zeros_like(acc)

        blk = jnp.sum(h_ref[...], axis=0, keepdims=True) / float(_N)
        acc[...] += jnp.broadcast_to(blk, acc.shape)

        @pl.when(i == nblk - 1)
        def _():
            hg = acc[...][0:1]
            x = _leaky(jnp.dot(hg, w1_ref[...], preferred_element_type=_F32)
                       + b1_ref[...])
            x = _leaky(jnp.dot(x, w2_ref[...], preferred_element_type=_F32)
                       + b2_ref[...])
            z1 = jnp.dot(x, wp1_ref[...], preferred_element_type=_F32) + bp1_ref[...]
            z2 = jnp.dot(x, wp2_ref[...], preferred_element_type=_F32) + bp2_ref[...]
            p1_ref[...] = 1.0 / (1.0 + jnp.exp(-z1))
            p2_ref[...] = 1.0 / (1.0 + jnp.exp(-z2))

    full = lambda a: pl.BlockSpec(a.shape, lambda i: tuple(0 for _ in a.shape))
    return pl.pallas_call(
        body,
        grid=(nblk,),
        in_specs=[pl.BlockSpec((B, 64), lambda i: (i, 0)),
                  full(w1), full(b1), full(w2), full(b2),
                  full(wp1), full(bp1), full(wp2), full(bp2)],
        out_specs=[pl.BlockSpec((1, 2), lambda i: (0, 0)),
                   pl.BlockSpec((1, 2), lambda i: (0, 0))],
        out_shape=[jax.ShapeDtypeStruct((1, 2), _F32),
                   jax.ShapeDtypeStruct((1, 2), _F32)],
        scratch_shapes=[pltpu.VMEM((8, 64), _F32)],
    )(h, w1, b1, w2, b2, wp1, bp1, wp2, bp2)


def _split_edges(edge_index):
    Br = _E // 128

    def body(ei_ref, s_ref, d_ref):
        s_ref[...] = ei_ref[0].reshape(Br, 128)
        d_ref[...] = ei_ref[1].reshape(Br, 128)

    return pl.pallas_call(
        body,
        out_shape=[jax.ShapeDtypeStruct((Br, 128), jnp.int32),
                   jax.ShapeDtypeStruct((Br, 128), jnp.int32)],
    )(edge_index)


def _sc_mesh():
    return plsc.VectorSubcoreMesh(core_axis_name="c", subcore_axis_name="s",
                                  num_cores=_NC, num_subcores=_NS)


def _sc_gather(src2, dst2, ntab, fnjb, half, e1=None, ew=None):
    lo = half * _GNCH
    with_e = e1 is not None

    scratch = [
        pltpu.VMEM((8, 128), jnp.int32),
        pltpu.VMEM((_GC, 128), _F32),
        pltpu.VMEM((_GC, 32), _F32),
        pltpu.SemaphoreType.DMA,
        pltpu.SemaphoreType.DMA,
        pltpu.SemaphoreType.DMA,
        pltpu.SemaphoreType.DMA,
    ]
    if with_e:
        scratch = scratch + [pltpu.VMEM((_GC + 16,), _F32),
                             pltpu.VMEM((128,), _F32)]

    @functools.partial(
        pl.kernel,
        out_type=jax.ShapeDtypeStruct((_EH, 128), _F32),
        mesh=_sc_mesh(),
        scratch_types=scratch,
        compiler_params=pltpu.CompilerParams(use_tc_tiling_on_sc=False),
    )
    def k(*refs):
        if with_e:
            (src_h, dst_h, ntab_h, fnjb_h, e_h, ew_h, og_h,
             idx, bufa, bufb, sema, semb, semi, semo, ebuf, wbuf) = refs
            pltpu.sync_copy(ew_h, wbuf)
        else:
            (src_h, dst_h, ntab_h, fnjb_h, og_h,
             idx, bufa, bufb, sema, semb, semi, semo) = refs
        wid = lax.axis_index("s") * _NC + lax.axis_index("c")

        def body(ci, carry):
            chunk = lo + ci * _NW + wid

            @pl.when(chunk < lo + _GNCH)
            def _():
                base = chunk * _GC
                row = chunk * 2
                cps = []
                for j in range(2):
                    cps.append(pltpu.async_copy(
                        src_h.at[row + j], idx.at[j], semi))
                    cps.append(pltpu.async_copy(
                        dst_h.at[row + j], idx.at[2 + j], semi))
                if with_e:
                    cps.append(pltpu.async_copy(
                        e_h.at[pl.ds(base, _GC)],
                        ebuf.at[pl.ds(0, _GC)], semi))
                for cp in cps:
                    cp.wait()

                @pl.when(ci > 0)
                def _():
                    pltpu.make_async_copy(
                        bufa, og_h.at[pl.ds(0, _GC)], semo).wait()

                cps = []
                for j in range(2):
                    cps.append(pltpu.async_copy(
                        ntab_h.at[idx.at[j]],
                        bufa.at[pl.ds(j * 128, 128)], sema))
                    cps.append(pltpu.async_copy(
                        fnjb_h.at[idx.at[2 + j]],
                        bufb.at[pl.ds(j * 128, 128)], semb))
                for cp in cps:
                    cp.wait()

                def addrow(r, carry2):
                    if with_e:
                        ev = ebuf[pl.ds(r, 16)][0]
                        bufa[r, pl.ds(0, 16)] += (bufb[r, pl.ds(0, 16)]
                                                  + ev * wbuf[pl.ds(0, 16)])
                        bufa[r, pl.ds(16, 16)] += (bufb[r, pl.ds(16, 16)]
                                                   + ev * wbuf[pl.ds(16, 16)])
                    else:
                        bufa[r, pl.ds(0, 16)] += bufb[r, pl.ds(0, 16)]
                        bufa[r, pl.ds(16, 16)] += bufb[r, pl.ds(16, 16)]
                    return carry2

                lax.fori_loop(0, _GC, addrow, 0)
                pltpu.async_copy(bufa, og_h.at[pl.ds(base - lo * _GC, _GC)],
                                 semo)

            return carry

        lax.fori_loop(0, _GPERW, body, 0)
        pltpu.make_async_copy(bufa, og_h.at[pl.ds(0, _GC)], semo).wait()

    if with_e:
        return k(src2, dst2, ntab, fnjb, e1, ew)
    return k(src2, dst2, ntab, fnjb)


def _sc_scatter(dst2, v128, half):
    nsub = _SC // 128
    clo = half * _SNCH

    @functools.partial(
        pl.kernel,
        out_type=jax.ShapeDtypeStruct((_N, 128), _F32),
        mesh=_sc_mesh(),
        scratch_types=[
            pltpu.VMEM((8, 128), jnp.int32),
            pltpu.VMEM((_SC, 16), _F32),
            pltpu.VMEM((_ZC, 16), _F32),
            pltpu.VMEM((_ZC, 16), _F32),
            pltpu.VMEM_SHARED((_N, 16), _F32),
            pltpu.SemaphoreType.DMA,
            pltpu.SemaphoreType.DMA,
        ],
        compiler_params=pltpu.CompilerParams(use_tc_tiling_on_sc=False),
    )
    def k(dst_h, v_h, acc_h, idx, vals, zbuf, obuf, table, semi, sems):
        c = lax.axis_index("c")
        s = lax.axis_index("s")
        row0 = s * _RPT

        def zb(i, carry):
            zbuf[i] = jnp.zeros((16,), _F32)
            return carry

        lax.fori_loop(0, _ZC, zb, 0)

        def one_round(b, lo, hi, ocol):
            for kk in range(_RPT // _ZC):
                pltpu.sync_copy(zbuf, table.at[pl.ds(row0 + kk * _ZC, _ZC)])
            plsc.subcore_barrier()

            def body(ci, carry):
                chunk = clo + ci * _NS + s

                @pl.when(jnp.logical_and(chunk >= lo, chunk < hi))
                def _():
                    base = (chunk - clo) * _SC
                    row = chunk * nsub
                    cps = []
                    for j in range(nsub):
                        cps.append(pltpu.async_copy(
                            dst_h.at[row + j], idx.at[j], semi))
                    pltpu.sync_copy(
                        v_h.at[pl.ds(base, _SC), pl.ds(16 * b, 16)], vals)
                    for cp in cps:
                        cp.wait()
                    cps = []
                    for j in range(nsub):
                        cps.append(pltpu.async_copy(
                            vals.at[pl.ds(j * 128, 128)],
                            table.at[idx.at[j]], sems, add=True))
                    for cp in cps:
                        cp.wait()

                return carry

            lax.fori_loop(0, _SPERT, body, 0)
            plsc.subcore_barrier()
            for kk in range(_RPT // _ZC):
                r0 = row0 + kk * _ZC
                pltpu.sync_copy(table.at[pl.ds(r0, _ZC)], obuf)
                pltpu.sync_copy(obuf, acc_h.at[pl.ds(r0, _ZC),
                                               pl.ds(ocol, 16)])

        mid = clo + _SNCH // 2
        for r in range(3):
            @pl.when(c == 0)
            def _(r=r):
                if r < 2:
                    one_round([0, 2][r], clo, clo + _SNCH, 16 * [0, 2][r])
                else:
                    one_round(4, clo, mid, 64)

            @pl.when(c == 1)
            def _(r=r):
                if r < 2:
                    one_round([1, 3][r], clo, clo + _SNCH, 16 * [1, 3][r])
                else:
                    one_round(4, mid, clo + _SNCH, 80)

    return k(dst2, v128)


def kernel(h, e, edge_index, params):
    src2, dst2 = _split_edges(edge_index)
    hp = jnp.pad(h, ((0, 0), (0, 2)))
    pw = jnp.pad(params["proj_h_W"], ((0, 2), (0, 0)))
    pb = params["proj_h_b"].reshape(1, 16)
    e1 = e.reshape(_E)

    h_cur = None
    e_half = [None, None]
    for li, (p, (in_n, in_e, out_n, out_e, H)) in enumerate(zip(params["layers"], _CFGS)):
        wall = jnp.concatenate([p["W_ni"], p["W_src"], p["W_nj"]], axis=1)
        brow = jnp.concatenate([jnp.zeros((1, 96), _F32),
                                p["b_e"].reshape(1, 32)], axis=1)
        if li == 0:
            ntab, fnjb = _node_table(hp, pw, pb, wall, brow, True)
            ew = jnp.pad((params["proj_e_W"] @ p["W_fij"]).reshape(32), (0, 96))
            bf = (params["proj_e_b"].reshape(1, 8) @ p["W_fij"]).reshape(1, 32)
            wf = None
        else:
            ntab, fnjb = _node_table(h_cur, pw, pb, wall, brow, False)
            ew = None
            wf = p["W_fij"]
            bf = jnp.zeros((1, 32), _F32)

        abd = jnp.zeros((32, 8), _F32)
        for hh in range(H):
            abd = abd.at[hh * out_e:(hh + 1) * out_e, hh].set(p["attn"][hh])
        smat = jnp.zeros((8, 64), _F32)
        for hh in range(H):
            smat = smat.at[hh, hh * out_n:(hh + 1) * out_n].set(1.0)
        pmat = jnp.zeros((8, 16), _F32)
        for hh in range(H):
            pmat = pmat.at[hh, hh].set(1.0)

        accs = [None, None]
        new_e = [None, None]
        for hf in range(2):
            if li == 0:
                og = _sc_gather(src2, dst2, ntab, fnjb, hf, e1, ew)
            else:
                og = _sc_gather(src2, dst2, ntab, fnjb, hf)
            f_out, v128 = _edge_math(e_half[hf], og, wf, bf, abd, smat, pmat)
            accs[hf] = _sc_scatter(dst2, v128, hf)
            new_e[hf] = f_out
        h_cur = _normalize(accs[0], accs[1], H)
        e_half = new_e

    pr = params
    return tuple(_readout(
        h_cur,
        pr["pred_W1"], pr["pred_b1"].reshape(1, 16),
        pr["pred_W2"], pr["pred_b2"].reshape(1, 8),
        pr["pred_Wp1"], pr["pred_bp1"].reshape(1, 2),
        pr["pred_Wp2"], pr["pred_bp2"].reshape(1, 2)))
